# baseline (speedup 1.0000x reference)
import jax
import jax.numpy as jnp
from jax.experimental import pallas as pl

_ANCHORS = ((1.08, 1.19), (3.42, 4.41), (6.63, 11.38), (9.42, 5.11), (16.62, 10.52))
_NA = 5
_NC = 7
_THR = 0.6
_H = 48
_W = 48
_TPAD = 56
_CHUNK = 128
_NCHUNK = (_H * _W) // _CHUNK
_SLOT = 16
_EPS = 1e-12

_HITC = _THR / (1.0 + _THR)

_DN = (((0,), (0,)), ((), ()))


def _region_loss_kernel(x_ref, t_ref, out_ref):
    t = t_ref[0]
    lab = t[:, 0:1]
    gx = t[:, 1:2] * float(_W)
    gy = t[:, 2:3] * float(_H)
    gw = t[:, 3:4] * float(_W)
    gl = t[:, 4:5] * float(_H)
    gim = t[:, 5:6]
    gre = t[:, 6:7]
    valid = t[:, 1:2] > 0.0
    validf = jnp.where(valid, 1.0, 0.0)
    gif = jnp.clip(jnp.floor(gx), 0.0, float(_W - 1))
    gjf = jnp.clip(jnp.floor(gy), 0.0, float(_H - 1))
    txv = gx - gif
    tyv = gy - gjf
    area_g = gw * gl

    best_v = jnp.full_like(gx, -1.0)
    best_n = jnp.zeros_like(gx)
    best_w = jnp.full_like(gx, _ANCHORS[0][0])
    best_h = jnp.full_like(gx, _ANCHORS[0][1])
    anch_iou = []
    for a, (aw, ah) in enumerate(_ANCHORS):
        inter = jnp.minimum(gw, aw) * jnp.minimum(gl, ah)
        iou = inter / (area_g + aw * ah - inter + 1e-16)
        anch_iou.append(iou)
        upd = iou > best_v
        best_v = jnp.where(upd, iou, best_v)
        best_n = jnp.where(upd, float(a), best_n)
        best_w = jnp.where(upd, aw, best_w)
        best_h = jnp.where(upd, ah, best_h)
    twv = jnp.log(gw / best_w + 1e-16)
    tlv = jnp.log(gl / best_h + 1e-16)

    labcl = jnp.clip(lab, 0.0, float(_NC - 1))
    tio_i = jax.lax.broadcasted_iota(jnp.int32, (_TPAD, 1), 0)
    pow2 = jax.lax.bitcast_convert_type((tio_i + 127) << 23, jnp.float32)
    dblpow = pow2 * 2.0

    ones_col = jnp.ones_like(gx)
    zero_col = jnp.zeros_like(gx)
    vals_cols = jnp.concatenate(
        [txv, tyv, twv, tlv, gim, gre, ones_col, zero_col], axis=1)

    hw = gw * 0.5
    hh = gl * 0.5
    b1x1 = gx - hw
    b1x2 = gx + hw
    b1y1 = gy - hh
    b1y2 = gy + hh
    ca_g = _HITC * area_g

    lane = jax.lax.broadcasted_iota(jnp.int32, (1, _CHUNK), 1).astype(jnp.float32)
    row_i = jax.lax.broadcasted_iota(jnp.int32, (8, 1), 0)
    sigmask = jnp.logical_or(row_i <= 1, row_i == 6)
    errmask = jnp.where(row_i < 6, 1.0, 0.0)
    clsmask = jnp.where(row_i < _NC, 1.0, 0.0)

    acc0 = jnp.zeros((1, _CHUNK), dtype=jnp.float32)
    carry0 = (acc0, acc0, acc0, acc0, acc0)

    pafs, a_colss = [], []
    for a in range(_NA):
        paf = jnp.where(jnp.logical_and(valid, best_n == float(a)), 1.0, 0.0)
        zff = jnp.where(jnp.logical_and(anch_iou[a] > _THR, valid), 1.0, 0.0)
        labf = [jnp.where(labcl == float(c), 1.0, 0.0) * paf for c in range(_NC)]
        pafs.append(paf)
        a_colss.append(jnp.concatenate(labf + [pow2 * paf, zff], axis=1))

    def chunk_contrib(k, carry):
        acc_obj, acc_cls, acc_conf, acc_ncm, acc_nobj = carry
        idx = k.astype(jnp.float32) * float(_CHUNK) + lane
        jcell = jnp.floor(idx * (1.0 / float(_W)))
        icell = idx - jcell * float(_W)
        cellm = jnp.where(
            jnp.logical_and(gif == icell, gjf == jcell), 1.0, 0.0)
        for a, (aw, ah) in enumerate(_ANCHORS):
            paf = pafs[a]
            a_cols = a_colss[a]
            base_s = a * _SLOT
            slab = x_ref[0, pl.ds(k, 1), base_s:base_s + 8, :][0]
            slabL = x_ref[0, pl.ds(k, 1), base_s + 8:base_s + 16, :][0]
            sig = 1.0 / (1.0 + jnp.exp(-slab))
            pv = jnp.where(sigmask, sig, slab)
            ep = jnp.exp(slab)
            px = pv[0:1, :]
            py = pv[1:2, :]
            conf = pv[6:7, :]
            bw = ep[2:3, :] * aw
            bh = ep[3:4, :] * ah
            bx = px + icell
            by = py + jcell
            b2x1 = bx - bw * 0.5
            b2x2 = bx + bw * 0.5
            b2y1 = by - bh * 0.5
            b2y2 = by + bh * 0.5
            a2 = bw * bh

            iw = jnp.maximum(jnp.minimum(b1x2, b2x2) - jnp.maximum(b1x1, b2x1), 0.0)
            ih = jnp.maximum(jnp.minimum(b1y2, b2y2) - jnp.maximum(b1y1, b2y1), 0.0)
            inter = iw * ih
            hitf = jnp.where(inter > ca_g + _HITC * a2, 1.0, 0.0)
            hitcnt = jax.lax.dot_general(validf, hitf, _DN,
                                         preferred_element_type=jnp.float32)

            mm1 = jax.lax.dot_general(a_cols, cellm, _DN,
                                      preferred_element_type=jnp.float32)
            s_pow = mm1[_NC:_NC + 1, :]
            s_z = mm1[_NC + 1:_NC + 2, :]
            w = cellm * paf * jnp.where(dblpow > s_pow, 1.0, 0.0)
            mm2 = jax.lax.dot_general(vals_cols, w, _DN,
                                      preferred_element_type=jnp.float32)
            anym = mm2[6:7, :]

            d = pv - mm2
            acc_obj = acc_obj + anym * jnp.sum(d * d * errmask,
                                               axis=0, keepdims=True)

            el = jnp.exp(slabL)
            lse = jnp.log(jnp.sum(el * clsmask, axis=0, keepdims=True))
            any8 = jnp.minimum(mm1[0:8, :], 1.0)
            acc_cls = acc_cls + jnp.sum(any8 * (lse - slabL) * clsmask,
                                        axis=0, keepdims=True)

            base = jnp.where(hitcnt > 0.0, 0.0, 1.0)
            cm = jnp.where(anym > 0.0, 1.0, jnp.where(s_z > 0.0, 0.0, base))
            bce = jnp.where(anym > 0.0, -jnp.log(conf + _EPS),
                            -jnp.log(1.0 - conf + _EPS))
            acc_conf = acc_conf + cm * bce
            acc_ncm = acc_ncm + cm
            acc_nobj = acc_nobj + anym
        return (acc_obj, acc_cls, acc_conf, acc_ncm, acc_nobj)

    def body(k3, carry):
        carry = chunk_contrib(k3 * 3, carry)
        carry = chunk_contrib(k3 * 3 + 1, carry)
        carry = chunk_contrib(k3 * 3 + 2, carry)
        return carry

    carry0 = jax.lax.fori_loop(0, _NCHUNK // 3, body, carry0)

    acc_obj, acc_cls, acc_conf, acc_ncm, acc_nobj = carry0
    out_ref[0] = jnp.concatenate(
        [acc_obj, acc_cls, acc_conf, acc_ncm, acc_nobj], axis=0)


def kernel(x, target):
    nB = x.shape[0]
    nT = target.shape[1]
    x5 = x.reshape(nB, _NA, 7 + _NC, _NCHUNK, _CHUNK)
    pad1 = ((0, 0), (0, 0), (0, 1), (0, 0), (0, 0))
    x5 = jnp.concatenate(
        [jnp.pad(x5[:, :, :7], pad1), jnp.pad(x5[:, :, 7:], pad1)], axis=2)
    xr = x5.transpose(0, 3, 1, 2, 4).reshape(nB, _NCHUNK, _NA * _SLOT, _CHUNK)
    tp = jnp.pad(target, ((0, 0), (0, _TPAD - nT), (0, 1)))
    out = pl.pallas_call(
        _region_loss_kernel,
        grid=(nB,),
        in_specs=[
            pl.BlockSpec((1, _NCHUNK, _NA * _SLOT, _CHUNK),
                         lambda b: (b, 0, 0, 0)),
            pl.BlockSpec((1, _TPAD, 8), lambda b: (b, 0, 0)),
        ],
        out_specs=pl.BlockSpec((1, 5, _CHUNK), lambda b: (b, 0, 0)),
        out_shape=jax.ShapeDtypeStruct((nB, 5, _CHUNK), jnp.float32),
    )(xr, tp)
    sums = jnp.sum(out, axis=(0, 2))
    n_obj = jnp.maximum(sums[4], 1.0)
    n_cm = jnp.maximum(sums[3], 1.0)
    return (sums[0] + sums[1]) / n_obj + sums[2] / n_cm


# speedup vs baseline: 1.0289x; 1.0289x over previous
import jax
import jax.numpy as jnp
from jax.experimental import pallas as pl

_ANCHORS = ((1.08, 1.19), (3.42, 4.41), (6.63, 11.38), (9.42, 5.11), (16.62, 10.52))
_NA = 5
_NC = 7
_THR = 0.6
_H = 48
_W = 48
_TPAD = 56
_CHUNK = 128
_NCHUNK = (_H * _W) // _CHUNK
_SLOT = 16
_EPS = 1e-12

_HITC = _THR / (1.0 + _THR)

_DN = (((0,), (0,)), ((), ()))


def _region_loss_kernel(x_ref, t_ref, out_ref):
    t = t_ref[0]
    lab = t[:, 0:1]
    gx = t[:, 1:2] * float(_W)
    gy = t[:, 2:3] * float(_H)
    gw = t[:, 3:4] * float(_W)
    gl = t[:, 4:5] * float(_H)
    gim = t[:, 5:6]
    gre = t[:, 6:7]
    valid = t[:, 1:2] > 0.0
    validf = jnp.where(valid, 1.0, 0.0)
    gif = jnp.clip(jnp.floor(gx), 0.0, float(_W - 1))
    gjf = jnp.clip(jnp.floor(gy), 0.0, float(_H - 1))
    txv = gx - gif
    tyv = gy - gjf
    area_g = gw * gl

    best_v = jnp.full_like(gx, -1.0)
    best_n = jnp.zeros_like(gx)
    best_w = jnp.full_like(gx, _ANCHORS[0][0])
    best_h = jnp.full_like(gx, _ANCHORS[0][1])
    anch_iou = []
    for a, (aw, ah) in enumerate(_ANCHORS):
        inter = jnp.minimum(gw, aw) * jnp.minimum(gl, ah)
        iou = inter / (area_g + aw * ah - inter + 1e-16)
        anch_iou.append(iou)
        upd = iou > best_v
        best_v = jnp.where(upd, iou, best_v)
        best_n = jnp.where(upd, float(a), best_n)
        best_w = jnp.where(upd, aw, best_w)
        best_h = jnp.where(upd, ah, best_h)
    twv = jnp.log(gw / best_w + 1e-16)
    tlv = jnp.log(gl / best_h + 1e-16)

    labcl = jnp.clip(lab, 0.0, float(_NC - 1))
    tio_i = jax.lax.broadcasted_iota(jnp.int32, (_TPAD, 1), 0)
    pow2 = jax.lax.bitcast_convert_type((tio_i + 127) << 23, jnp.float32)
    dblpow = pow2 * 2.0

    ones_col = jnp.ones_like(gx)
    zero_col = jnp.zeros_like(gx)
    vals_cols = jnp.concatenate(
        [txv, tyv, twv, tlv, gim, gre, ones_col, zero_col], axis=1)

    hw = gw * 0.5
    hh = gl * 0.5
    b1x1 = gx - hw
    b1x2 = gx + hw
    b1y1 = gy - hh
    b1y2 = gy + hh
    ca_g = _HITC * area_g

    lane = jax.lax.broadcasted_iota(jnp.int32, (1, _CHUNK), 1).astype(jnp.float32)
    row_i = jax.lax.broadcasted_iota(jnp.int32, (8, 1), 0)
    sigmask = jnp.logical_or(row_i <= 1, row_i == 6)
    errmask = jnp.where(row_i < 6, 1.0, 0.0)
    clsmask = jnp.where(row_i < _NC, 1.0, 0.0)

    acc0 = jnp.zeros((1, _CHUNK), dtype=jnp.float32)
    carry0 = (acc0, acc0, acc0, acc0, acc0)

    pafs, a_colss = [], []
    for a in range(_NA):
        paf = jnp.where(jnp.logical_and(valid, best_n == float(a)), 1.0, 0.0)
        zff = jnp.where(jnp.logical_and(anch_iou[a] > _THR, valid), 1.0, 0.0)
        labf = [jnp.where(labcl == float(c), 1.0, 0.0) * paf for c in range(_NC)]
        pafs.append(paf)
        a_colss.append(jnp.concatenate(labf + [pow2 * paf, zff], axis=1))

    def chunk_contrib(k, carry):
        acc_obj, acc_cls, acc_conf, acc_ncm, acc_nobj = carry
        idx = k.astype(jnp.float32) * float(_CHUNK) + lane
        jcell = jnp.floor(idx * (1.0 / float(_W)))
        icell = idx - jcell * float(_W)
        cellm = jnp.where(
            jnp.logical_and(gif == icell, gjf == jcell), 1.0, 0.0)
        for a, (aw, ah) in enumerate(_ANCHORS):
            paf = pafs[a]
            a_cols = a_colss[a]
            base_s = a * _SLOT
            slab = x_ref[0, pl.ds(k, 1), base_s:base_s + 8, :][0]
            slabL = x_ref[0, pl.ds(k, 1), base_s + 8:base_s + 16, :][0]
            sig = 1.0 / (1.0 + jnp.exp(-slab))
            pv = jnp.where(sigmask, sig, slab)
            ep = jnp.exp(slab)
            px = pv[0:1, :]
            py = pv[1:2, :]
            conf = pv[6:7, :]
            bw = ep[2:3, :] * aw
            bh = ep[3:4, :] * ah
            bx = px + icell
            by = py + jcell
            b2x1 = bx - bw * 0.5
            b2x2 = bx + bw * 0.5
            b2y1 = by - bh * 0.5
            b2y2 = by + bh * 0.5
            a2 = bw * bh

            iw = jnp.maximum(jnp.minimum(b1x2, b2x2) - jnp.maximum(b1x1, b2x1), 0.0)
            ih = jnp.maximum(jnp.minimum(b1y2, b2y2) - jnp.maximum(b1y1, b2y1), 0.0)
            inter = iw * ih
            hitf = jnp.where(inter > ca_g + _HITC * a2, 1.0, 0.0)
            hitcnt = jax.lax.dot_general(validf, hitf, _DN,
                                         preferred_element_type=jnp.float32)

            mm1 = jax.lax.dot_general(a_cols, cellm, _DN,
                                      preferred_element_type=jnp.float32)
            s_pow = mm1[_NC:_NC + 1, :]
            s_z = mm1[_NC + 1:_NC + 2, :]
            w = cellm * paf * jnp.where(dblpow > s_pow, 1.0, 0.0)
            mm2 = jax.lax.dot_general(vals_cols, w, _DN,
                                      preferred_element_type=jnp.float32)
            anym = mm2[6:7, :]

            d = pv - mm2
            acc_obj = acc_obj + anym * jnp.sum(d * d * errmask,
                                               axis=0, keepdims=True)

            el = jnp.exp(slabL)
            lse = jnp.log(jnp.sum(el * clsmask, axis=0, keepdims=True))
            any8 = jnp.minimum(mm1[0:8, :], 1.0)
            acc_cls = acc_cls + jnp.sum(any8 * (lse - slabL) * clsmask,
                                        axis=0, keepdims=True)

            base = jnp.where(hitcnt > 0.0, 0.0, 1.0)
            cm = jnp.where(anym > 0.0, 1.0, jnp.where(s_z > 0.0, 0.0, base))
            bce = jnp.where(anym > 0.0, -jnp.log(conf + _EPS),
                            -jnp.log(1.0 - conf + _EPS))
            acc_conf = acc_conf + cm * bce
            acc_ncm = acc_ncm + cm
            acc_nobj = acc_nobj + anym
        return (acc_obj, acc_cls, acc_conf, acc_ncm, acc_nobj)

    def body(k6, carry):
        for u in range(6):
            carry = chunk_contrib(k6 * 6 + u, carry)
        return carry

    carry0 = jax.lax.fori_loop(0, _NCHUNK // 6, body, carry0)

    acc_obj, acc_cls, acc_conf, acc_ncm, acc_nobj = carry0
    out_ref[0] = jnp.concatenate(
        [acc_obj, acc_cls, acc_conf, acc_ncm, acc_nobj], axis=0)


def kernel(x, target):
    nB = x.shape[0]
    nT = target.shape[1]
    x5 = x.reshape(nB, _NA, 7 + _NC, _NCHUNK, _CHUNK)
    pad1 = ((0, 0), (0, 0), (0, 1), (0, 0), (0, 0))
    x5 = jnp.concatenate(
        [jnp.pad(x5[:, :, :7], pad1), jnp.pad(x5[:, :, 7:], pad1)], axis=2)
    xr = x5.transpose(0, 3, 1, 2, 4).reshape(nB, _NCHUNK, _NA * _SLOT, _CHUNK)
    tp = jnp.pad(target, ((0, 0), (0, _TPAD - nT), (0, 1)))
    out = pl.pallas_call(
        _region_loss_kernel,
        grid=(nB,),
        in_specs=[
            pl.BlockSpec((1, _NCHUNK, _NA * _SLOT, _CHUNK),
                         lambda b: (b, 0, 0, 0)),
            pl.BlockSpec((1, _TPAD, 8), lambda b: (b, 0, 0)),
        ],
        out_specs=pl.BlockSpec((1, 5, _CHUNK), lambda b: (b, 0, 0)),
        out_shape=jax.ShapeDtypeStruct((nB, 5, _CHUNK), jnp.float32),
    )(xr, tp)
    sums = jnp.sum(out, axis=(0, 2))
    n_obj = jnp.maximum(sums[4], 1.0)
    n_cm = jnp.maximum(sums[3], 1.0)
    return (sums[0] + sums[1]) / n_obj + sums[2] / n_cm


# speedup vs baseline: 1.3513x; 1.3134x over previous
import jax
import jax.numpy as jnp
import numpy as np
from jax.experimental import pallas as pl

_ANCHORS = ((1.08, 1.19), (3.42, 4.41), (6.63, 11.38), (9.42, 5.11), (16.62, 10.52))
_NA = 5
_NC = 7
_THR = 0.6
_H = 48
_W = 48
_TPAD = 56
_CHUNK = 128
_NCHUNK = (_H * _W) // _CHUNK
_EPS = 1e-12


_HITC = _THR / (1.0 + _THR)

_DN = (((0,), (0,)), ((), ()))


def _region_loss_kernel(x_ref, t_ref, out_ref):
    t = t_ref[0]
    lab = t[:, 0:1]
    gx = t[:, 1:2] * float(_W)
    gy = t[:, 2:3] * float(_H)
    gw = t[:, 3:4] * float(_W)
    gl = t[:, 4:5] * float(_H)
    gim = t[:, 5:6]
    gre = t[:, 6:7]
    valid = t[:, 1:2] > 0.0
    validf = jnp.where(valid, 1.0, 0.0)
    gif = jnp.clip(jnp.floor(gx), 0.0, float(_W - 1))
    gjf = jnp.clip(jnp.floor(gy), 0.0, float(_H - 1))
    txv = gx - gif
    tyv = gy - gjf
    area_g = gw * gl

    best_v = jnp.full_like(gx, -1.0)
    best_n = jnp.zeros_like(gx)
    best_w = jnp.full_like(gx, _ANCHORS[0][0])
    best_h = jnp.full_like(gx, _ANCHORS[0][1])
    anch_iou = []
    for a, (aw, ah) in enumerate(_ANCHORS):
        inter = jnp.minimum(gw, aw) * jnp.minimum(gl, ah)
        iou = inter / (area_g + aw * ah - inter + 1e-16)
        anch_iou.append(iou)
        upd = iou > best_v
        best_v = jnp.where(upd, iou, best_v)
        best_n = jnp.where(upd, float(a), best_n)
        best_w = jnp.where(upd, aw, best_w)
        best_h = jnp.where(upd, ah, best_h)
    twv = jnp.log(gw / best_w + 1e-16)
    tlv = jnp.log(gl / best_h + 1e-16)

    labcl = jnp.clip(lab, 0.0, float(_NC - 1))
    tio_i = jax.lax.broadcasted_iota(jnp.int32, (_TPAD, 1), 0)
    pow2 = jax.lax.bitcast_convert_type((tio_i + 127) << 23, jnp.float32)
    dblpow = pow2 * 2.0

    ones_col = jnp.ones_like(gx)
    vals_cols = jnp.concatenate(
        [ones_col, txv, tyv, twv, tlv, gim, gre], axis=1)

    hw = gw * 0.5
    hh = gl * 0.5
    b1x1 = gx - hw
    b1x2 = gx + hw
    b1y1 = gy - hh
    b1y2 = gy + hh

    lane = jax.lax.broadcasted_iota(jnp.int32, (1, _CHUNK), 1).astype(jnp.float32)

    acc0 = jnp.zeros((1, _CHUNK), dtype=jnp.float32)
    carry0 = (acc0, acc0, acc0, acc0, acc0)

    pafs, a_colss = [], []
    for a in range(_NA):
        paf = jnp.where(jnp.logical_and(valid, best_n == float(a)), 1.0, 0.0)
        zff = jnp.where(jnp.logical_and(anch_iou[a] > _THR, valid), 1.0, 0.0)
        labf = [jnp.where(labcl == float(c), 1.0, 0.0) * paf for c in range(_NC)]
        pafs.append(paf)
        a_colss.append(jnp.concatenate([pow2 * paf, zff] + labf, axis=1))

    ca_g = _HITC * area_g

    def chunk_contrib(k, carry):
        acc_obj, acc_cls, acc_conf, acc_ncm, acc_nobj = carry
        idx = k.astype(jnp.float32) * float(_CHUNK) + lane
        jcell = jnp.floor(idx * (1.0 / float(_W)))
        icell = idx - jcell * float(_W)
        cellm = jnp.where(
            jnp.logical_and(gif == icell, gjf == jcell), 1.0, 0.0)
        for a, (aw, ah) in enumerate(_ANCHORS):
            paf = pafs[a]
            a_cols = a_colss[a]
            base_c = a * (7 + _NC)
            px = jax.nn.sigmoid(x_ref[0, base_c + 0, pl.ds(k, 1), :])
            py = jax.nn.sigmoid(x_ref[0, base_c + 1, pl.ds(k, 1), :])
            pw = x_ref[0, base_c + 2, pl.ds(k, 1), :]
            ph = x_ref[0, base_c + 3, pl.ds(k, 1), :]
            pim = x_ref[0, base_c + 4, pl.ds(k, 1), :]
            pre = x_ref[0, base_c + 5, pl.ds(k, 1), :]
            conf = jax.nn.sigmoid(x_ref[0, base_c + 6, pl.ds(k, 1), :])

            bw = jnp.exp(pw) * aw
            bh = jnp.exp(ph) * ah
            bx = px + icell
            by = py + jcell
            b2x1 = bx - bw * 0.5
            b2x2 = bx + bw * 0.5
            b2y1 = by - bh * 0.5
            b2y2 = by + bh * 0.5
            a2 = bw * bh

            iw = jnp.maximum(jnp.minimum(b1x2, b2x2) - jnp.maximum(b1x1, b2x1), 0.0)
            ih = jnp.maximum(jnp.minimum(b1y2, b2y2) - jnp.maximum(b1y1, b2y1), 0.0)
            inter = iw * ih
            hitf = jnp.where(inter > ca_g + _HITC * a2, 1.0, 0.0)
            hitcnt = jax.lax.dot_general(validf, hitf, _DN,
                                         preferred_element_type=jnp.float32)

            mm1 = jax.lax.dot_general(a_cols, cellm, _DN,
                                      preferred_element_type=jnp.float32)
            s_pow = mm1[0:1, :]
            s_z = mm1[1:2, :]
            w = cellm * paf * jnp.where(dblpow > s_pow, 1.0, 0.0)
            mm2 = jax.lax.dot_general(vals_cols, w, _DN,
                                      preferred_element_type=jnp.float32)
            anym = mm2[0:1, :]

            d = px - mm2[1:2, :]
            err = d * d
            d = py - mm2[2:3, :]
            err = err + d * d
            d = pw - mm2[3:4, :]
            err = err + d * d
            d = ph - mm2[4:5, :]
            err = err + d * d
            d = pim - mm2[5:6, :]
            err = err + d * d
            d = pre - mm2[6:7, :]
            err = err + d * d
            acc_obj = acc_obj + anym * err

            logits = [x_ref[0, base_c + 7 + c, pl.ds(k, 1), :] for c in range(_NC)]
            m = logits[0]
            for c in range(1, _NC):
                m = jnp.maximum(m, logits[c])
            s = jnp.exp(logits[0] - m)
            for c in range(1, _NC):
                s = s + jnp.exp(logits[c] - m)
            lse = jnp.log(s) + m
            for c in range(_NC):
                anyc = jnp.minimum(mm1[2 + c:3 + c, :], 1.0)
                acc_cls = acc_cls + anyc * (lse - logits[c])

            base = jnp.where(hitcnt > 0.0, 0.0, 1.0)
            cm = jnp.where(anym > 0.0, 1.0, jnp.where(s_z > 0.0, 0.0, base))
            bce = jnp.where(anym > 0.0, -jnp.log(conf + _EPS),
                            -jnp.log(1.0 - conf + _EPS))
            acc_conf = acc_conf + cm * bce
            acc_ncm = acc_ncm + cm
            acc_nobj = acc_nobj + anym
        return (acc_obj, acc_cls, acc_conf, acc_ncm, acc_nobj)

    def body(k6, carry):
        for u in range(6):
            carry = chunk_contrib(k6 * 6 + u, carry)
        return carry

    carry0 = jax.lax.fori_loop(0, _NCHUNK // 6, body, carry0)

    acc_obj, acc_cls, acc_conf, acc_ncm, acc_nobj = carry0
    out_ref[0] = jnp.concatenate(
        [acc_obj, acc_cls, acc_conf, acc_ncm, acc_nobj], axis=0)


def kernel(x, target):
    nB = x.shape[0]
    nT = target.shape[1]
    xr = x.reshape(nB, _NA * (7 + _NC), _NCHUNK, _CHUNK)
    tp = jnp.pad(target, ((0, 0), (0, _TPAD - nT), (0, 1)))
    out = pl.pallas_call(
        _region_loss_kernel,
        grid=(nB,),
        in_specs=[
            pl.BlockSpec((1, _NA * (7 + _NC), _NCHUNK, _CHUNK),
                         lambda b: (b, 0, 0, 0)),
            pl.BlockSpec((1, _TPAD, 8), lambda b: (b, 0, 0)),
        ],
        out_specs=pl.BlockSpec((1, 5, _CHUNK), lambda b: (b, 0, 0)),
        out_shape=jax.ShapeDtypeStruct((nB, 5, _CHUNK), jnp.float32),
    )(xr, tp)
    sums = jnp.sum(out, axis=(0, 2))
    n_obj = jnp.maximum(sums[4], 1.0)
    n_cm = jnp.maximum(sums[3], 1.0)
    return (sums[0] + sums[1]) / n_obj + sums[2] / n_cm


# speedup vs baseline: 1.4023x; 1.0377x over previous
import jax
import jax.numpy as jnp
import numpy as np
from jax.experimental import pallas as pl

_ANCHORS = ((1.08, 1.19), (3.42, 4.41), (6.63, 11.38), (9.42, 5.11), (16.62, 10.52))
_NA = 5
_NC = 7
_THR = 0.6
_H = 48
_W = 48
_TPAD = 56
_CHUNK = 128
_NCHUNK = (_H * _W) // _CHUNK
_EPS = 1e-12


_HITC = _THR / (1.0 + _THR)

_DN = (((0,), (0,)), ((), ()))


def _region_loss_kernel(x_ref, t_ref, out_ref):
    t = t_ref[0]
    lab = t[:, 0:1]
    gx = t[:, 1:2] * float(_W)
    gy = t[:, 2:3] * float(_H)
    gw = t[:, 3:4] * float(_W)
    gl = t[:, 4:5] * float(_H)
    gim = t[:, 5:6]
    gre = t[:, 6:7]
    valid = t[:, 1:2] > 0.0
    validf = jnp.where(valid, 1.0, 0.0)
    gif = jnp.clip(jnp.floor(gx), 0.0, float(_W - 1))
    gjf = jnp.clip(jnp.floor(gy), 0.0, float(_H - 1))
    txv = gx - gif
    tyv = gy - gjf
    area_g = gw * gl

    best_v = jnp.full_like(gx, -1.0)
    best_n = jnp.zeros_like(gx)
    best_w = jnp.full_like(gx, _ANCHORS[0][0])
    best_h = jnp.full_like(gx, _ANCHORS[0][1])
    anch_iou = []
    for a, (aw, ah) in enumerate(_ANCHORS):
        inter = jnp.minimum(gw, aw) * jnp.minimum(gl, ah)
        iou = inter / (area_g + aw * ah - inter + 1e-16)
        anch_iou.append(iou)
        upd = iou > best_v
        best_v = jnp.where(upd, iou, best_v)
        best_n = jnp.where(upd, float(a), best_n)
        best_w = jnp.where(upd, aw, best_w)
        best_h = jnp.where(upd, ah, best_h)
    twv = jnp.log(gw / best_w + 1e-16)
    tlv = jnp.log(gl / best_h + 1e-16)

    labcl = jnp.clip(lab, 0.0, float(_NC - 1))
    tio_i = jax.lax.broadcasted_iota(jnp.int32, (_TPAD, 1), 0)
    pow2 = jax.lax.bitcast_convert_type((tio_i + 127) << 23, jnp.float32)
    dblpow = pow2 * 2.0

    ones_col = jnp.ones_like(gx)
    vals_cols = jnp.concatenate(
        [ones_col, txv, tyv, twv, tlv, gim, gre], axis=1)

    hw = gw * 0.5
    hh = gl * 0.5
    b1x1 = gx - hw
    b1x2 = gx + hw
    b1y1 = gy - hh
    b1y2 = gy + hh

    lane = jax.lax.broadcasted_iota(jnp.int32, (1, _CHUNK), 1).astype(jnp.float32)

    acc0 = jnp.zeros((1, _CHUNK), dtype=jnp.float32)
    carry0 = (acc0, acc0, acc0, acc0, acc0)

    pafs, a_colss = [], []
    for a in range(_NA):
        paf = jnp.where(jnp.logical_and(valid, best_n == float(a)), 1.0, 0.0)
        zff = jnp.where(jnp.logical_and(anch_iou[a] > _THR, valid), 1.0, 0.0)
        labf = [jnp.where(labcl == float(c), 1.0, 0.0) * paf for c in range(_NC)]
        pafs.append(paf)
        a_colss.append(jnp.concatenate([pow2 * paf, zff] + labf, axis=1))

    ca_g = _HITC * area_g

    def chunk_contrib(k, carry):
        acc_obj, acc_cls, acc_conf, acc_ncm, acc_nobj = carry
        idx = float(k * _CHUNK) + lane
        jcell = jnp.floor(idx * (1.0 / float(_W)))
        icell = idx - jcell * float(_W)
        cellm = jnp.where(
            jnp.logical_and(gif == icell, gjf == jcell), 1.0, 0.0)
        for a, (aw, ah) in enumerate(_ANCHORS):
            paf = pafs[a]
            a_cols = a_colss[a]
            base_c = a * (7 + _NC)
            px = jax.nn.sigmoid(x_ref[0, base_c + 0, pl.ds(k, 1), :])
            py = jax.nn.sigmoid(x_ref[0, base_c + 1, pl.ds(k, 1), :])
            pw = x_ref[0, base_c + 2, pl.ds(k, 1), :]
            ph = x_ref[0, base_c + 3, pl.ds(k, 1), :]
            pim = x_ref[0, base_c + 4, pl.ds(k, 1), :]
            pre = x_ref[0, base_c + 5, pl.ds(k, 1), :]
            conf = jax.nn.sigmoid(x_ref[0, base_c + 6, pl.ds(k, 1), :])

            bw = jnp.exp(pw) * aw
            bh = jnp.exp(ph) * ah
            bx = px + icell
            by = py + jcell
            b2x1 = bx - bw * 0.5
            b2x2 = bx + bw * 0.5
            b2y1 = by - bh * 0.5
            b2y2 = by + bh * 0.5
            a2 = bw * bh

            iw = jnp.maximum(jnp.minimum(b1x2, b2x2) - jnp.maximum(b1x1, b2x1), 0.0)
            ih = jnp.maximum(jnp.minimum(b1y2, b2y2) - jnp.maximum(b1y1, b2y1), 0.0)
            inter = iw * ih
            hitf = jnp.where(inter > ca_g + _HITC * a2, 1.0, 0.0)
            hitcnt = jax.lax.dot_general(validf, hitf, _DN,
                                         preferred_element_type=jnp.float32)

            mm1 = jax.lax.dot_general(a_cols, cellm, _DN,
                                      preferred_element_type=jnp.float32)
            s_pow = mm1[0:1, :]
            s_z = mm1[1:2, :]
            w = cellm * paf * jnp.where(dblpow > s_pow, 1.0, 0.0)
            mm2 = jax.lax.dot_general(vals_cols, w, _DN,
                                      preferred_element_type=jnp.float32)
            anym = mm2[0:1, :]

            d = px - mm2[1:2, :]
            err = d * d
            d = py - mm2[2:3, :]
            err = err + d * d
            d = pw - mm2[3:4, :]
            err = err + d * d
            d = ph - mm2[4:5, :]
            err = err + d * d
            d = pim - mm2[5:6, :]
            err = err + d * d
            d = pre - mm2[6:7, :]
            err = err + d * d
            acc_obj = acc_obj + anym * err

            logits = [x_ref[0, base_c + 7 + c, pl.ds(k, 1), :] for c in range(_NC)]
            m = logits[0]
            for c in range(1, _NC):
                m = jnp.maximum(m, logits[c])
            s = jnp.exp(logits[0] - m)
            for c in range(1, _NC):
                s = s + jnp.exp(logits[c] - m)
            lse = jnp.log(s) + m
            for c in range(_NC):
                anyc = jnp.minimum(mm1[2 + c:3 + c, :], 1.0)
                acc_cls = acc_cls + anyc * (lse - logits[c])

            base = jnp.where(hitcnt > 0.0, 0.0, 1.0)
            cm = jnp.where(anym > 0.0, 1.0, jnp.where(s_z > 0.0, 0.0, base))
            bce = jnp.where(anym > 0.0, -jnp.log(conf + _EPS),
                            -jnp.log(1.0 - conf + _EPS))
            acc_conf = acc_conf + cm * bce
            acc_ncm = acc_ncm + cm
            acc_nobj = acc_nobj + anym
        return (acc_obj, acc_cls, acc_conf, acc_ncm, acc_nobj)

    for kk in range(_NCHUNK):
        carry0 = chunk_contrib(kk, carry0)

    acc_obj, acc_cls, acc_conf, acc_ncm, acc_nobj = carry0
    out_ref[0] = jnp.concatenate(
        [acc_obj, acc_cls, acc_conf, acc_ncm, acc_nobj], axis=0)


def kernel(x, target):
    nB = x.shape[0]
    nT = target.shape[1]
    xr = x.reshape(nB, _NA * (7 + _NC), _NCHUNK, _CHUNK)
    tp = jnp.pad(target, ((0, 0), (0, _TPAD - nT), (0, 1)))
    out = pl.pallas_call(
        _region_loss_kernel,
        grid=(nB,),
        in_specs=[
            pl.BlockSpec((1, _NA * (7 + _NC), _NCHUNK, _CHUNK),
                         lambda b: (b, 0, 0, 0)),
            pl.BlockSpec((1, _TPAD, 8), lambda b: (b, 0, 0)),
        ],
        out_specs=pl.BlockSpec((1, 5, _CHUNK), lambda b: (b, 0, 0)),
        out_shape=jax.ShapeDtypeStruct((nB, 5, _CHUNK), jnp.float32),
    )(xr, tp)
    sums = jnp.sum(out, axis=(0, 2))
    n_obj = jnp.maximum(sums[4], 1.0)
    n_cm = jnp.maximum(sums[3], 1.0)
    return (sums[0] + sums[1]) / n_obj + sums[2] / n_cm


# speedup vs baseline: 1.4027x; 1.0003x over previous
import jax
import jax.numpy as jnp
import numpy as np
from jax.experimental import pallas as pl

_ANCHORS = ((1.08, 1.19), (3.42, 4.41), (6.63, 11.38), (9.42, 5.11), (16.62, 10.52))
_NA = 5
_NC = 7
_THR = 0.6
_H = 48
_W = 48
_TPAD = 56
_CHUNK = 128
_NCHUNK = (_H * _W) // _CHUNK
_EPS = 1e-12


_HITC = _THR / (1.0 + _THR)

_DN = (((0,), (0,)), ((), ()))


def _region_loss_kernel(x_ref, t_ref, out_ref):
    t = t_ref[0]
    lab = t[:, 0:1]
    gx = t[:, 1:2] * float(_W)
    gy = t[:, 2:3] * float(_H)
    gw = t[:, 3:4] * float(_W)
    gl = t[:, 4:5] * float(_H)
    gim = t[:, 5:6]
    gre = t[:, 6:7]
    valid = t[:, 1:2] > 0.0
    validf = jnp.where(valid, 1.0, 0.0)
    gif = jnp.clip(jnp.floor(gx), 0.0, float(_W - 1))
    gjf = jnp.clip(jnp.floor(gy), 0.0, float(_H - 1))
    txv = gx - gif
    tyv = gy - gjf
    area_g = gw * gl

    best_v = jnp.full_like(gx, -1.0)
    best_n = jnp.zeros_like(gx)
    best_w = jnp.full_like(gx, _ANCHORS[0][0])
    best_h = jnp.full_like(gx, _ANCHORS[0][1])
    anch_iou = []
    for a, (aw, ah) in enumerate(_ANCHORS):
        inter = jnp.minimum(gw, aw) * jnp.minimum(gl, ah)
        iou = inter / (area_g + aw * ah - inter + 1e-16)
        anch_iou.append(iou)
        upd = iou > best_v
        best_v = jnp.where(upd, iou, best_v)
        best_n = jnp.where(upd, float(a), best_n)
        best_w = jnp.where(upd, aw, best_w)
        best_h = jnp.where(upd, ah, best_h)
    twv = jnp.log(gw / best_w + 1e-16)
    tlv = jnp.log(gl / best_h + 1e-16)

    labcl = jnp.clip(lab, 0.0, float(_NC - 1))
    tio_i = jax.lax.broadcasted_iota(jnp.int32, (_TPAD, 1), 0)
    pow2 = jax.lax.bitcast_convert_type((tio_i + 127) << 23, jnp.float32)
    dblpow = pow2 * 2.0

    ones_col = jnp.ones_like(gx)
    vals_cols = jnp.concatenate(
        [ones_col, txv, tyv, twv, tlv, gim, gre], axis=1)

    hw = gw * 0.5
    hh = gl * 0.5
    b1x1 = gx - hw
    b1x2 = gx + hw
    b1y1 = gy - hh
    b1y2 = gy + hh

    lane = jax.lax.broadcasted_iota(jnp.int32, (1, _CHUNK), 1).astype(jnp.float32)

    acc0 = jnp.zeros((1, _CHUNK), dtype=jnp.float32)
    carry0 = (acc0, acc0, acc0, acc0, acc0)

    pafs, a_colss = [], []
    for a in range(_NA):
        paf = jnp.where(jnp.logical_and(valid, best_n == float(a)), 1.0, 0.0)
        zff = jnp.where(jnp.logical_and(anch_iou[a] > _THR, valid), 1.0, 0.0)
        labf = [jnp.where(labcl == float(c), 1.0, 0.0) * paf for c in range(_NC)]
        pafs.append(paf)
        a_colss.append(jnp.concatenate([pow2 * paf, zff] + labf, axis=1))

    ca_g = _HITC * area_g

    amaps = []
    for a, (aw, ah) in enumerate(_ANCHORS):
        base_c = a * (7 + _NC)
        px18 = jax.nn.sigmoid(x_ref[0, base_c + 0])
        py18 = jax.nn.sigmoid(x_ref[0, base_c + 1])
        bw18 = jnp.exp(x_ref[0, base_c + 2]) * aw
        bh18 = jnp.exp(x_ref[0, base_c + 3]) * ah
        conf18 = jax.nn.sigmoid(x_ref[0, base_c + 6])
        bcep18 = -jnp.log(conf18 + _EPS)
        bcen18 = -jnp.log(1.0 - conf18 + _EPS)
        lt = [x_ref[0, base_c + 7 + c] for c in range(_NC)]
        mx = lt[0]
        for c in range(1, _NC):
            mx = jnp.maximum(mx, lt[c])
        sm = jnp.exp(lt[0] - mx)
        for c in range(1, _NC):
            sm = sm + jnp.exp(lt[c] - mx)
        lse18 = jnp.log(sm) + mx
        amaps.append((px18, py18, bw18, bh18, bcep18, bcen18, lse18))

    def chunk_contrib(k, carry):
        acc_obj, acc_cls, acc_conf, acc_ncm, acc_nobj = carry
        idx = float(k * _CHUNK) + lane
        jcell = jnp.floor(idx * (1.0 / float(_W)))
        icell = idx - jcell * float(_W)
        cellm = jnp.where(
            jnp.logical_and(gif == icell, gjf == jcell), 1.0, 0.0)
        for a, (aw, ah) in enumerate(_ANCHORS):
            paf = pafs[a]
            a_cols = a_colss[a]
            base_c = a * (7 + _NC)
            px18, py18, bw18, bh18, bcep18, bcen18, lse18 = amaps[a]
            px = px18[k:k + 1, :]
            py = py18[k:k + 1, :]
            pw = x_ref[0, base_c + 2, pl.ds(k, 1), :]
            ph = x_ref[0, base_c + 3, pl.ds(k, 1), :]
            pim = x_ref[0, base_c + 4, pl.ds(k, 1), :]
            pre = x_ref[0, base_c + 5, pl.ds(k, 1), :]

            bw = bw18[k:k + 1, :]
            bh = bh18[k:k + 1, :]
            bx = px + icell
            by = py + jcell
            b2x1 = bx - bw * 0.5
            b2x2 = bx + bw * 0.5
            b2y1 = by - bh * 0.5
            b2y2 = by + bh * 0.5
            a2 = bw * bh

            iw = jnp.maximum(jnp.minimum(b1x2, b2x2) - jnp.maximum(b1x1, b2x1), 0.0)
            ih = jnp.maximum(jnp.minimum(b1y2, b2y2) - jnp.maximum(b1y1, b2y1), 0.0)
            inter = iw * ih
            hitf = jnp.where(inter > ca_g + _HITC * a2, 1.0, 0.0)
            hitcnt = jax.lax.dot_general(validf, hitf, _DN,
                                         preferred_element_type=jnp.float32)

            mm1 = jax.lax.dot_general(a_cols, cellm, _DN,
                                      preferred_element_type=jnp.float32)
            s_pow = mm1[0:1, :]
            s_z = mm1[1:2, :]
            w = cellm * paf * jnp.where(dblpow > s_pow, 1.0, 0.0)
            mm2 = jax.lax.dot_general(vals_cols, w, _DN,
                                      preferred_element_type=jnp.float32)
            anym = mm2[0:1, :]

            d = px - mm2[1:2, :]
            err = d * d
            d = py - mm2[2:3, :]
            err = err + d * d
            d = pw - mm2[3:4, :]
            err = err + d * d
            d = ph - mm2[4:5, :]
            err = err + d * d
            d = pim - mm2[5:6, :]
            err = err + d * d
            d = pre - mm2[6:7, :]
            err = err + d * d
            acc_obj = acc_obj + anym * err

            lse = lse18[k:k + 1, :]
            for c in range(_NC):
                anyc = jnp.minimum(mm1[2 + c:3 + c, :], 1.0)
                logit = x_ref[0, base_c + 7 + c, pl.ds(k, 1), :]
                acc_cls = acc_cls + anyc * (lse - logit)

            base = jnp.where(hitcnt > 0.0, 0.0, 1.0)
            cm = jnp.where(anym > 0.0, 1.0, jnp.where(s_z > 0.0, 0.0, base))
            bce = jnp.where(anym > 0.0, bcep18[k:k + 1, :], bcen18[k:k + 1, :])
            acc_conf = acc_conf + cm * bce
            acc_ncm = acc_ncm + cm
            acc_nobj = acc_nobj + anym
        return (acc_obj, acc_cls, acc_conf, acc_ncm, acc_nobj)

    for kk in range(_NCHUNK):
        carry0 = chunk_contrib(kk, carry0)

    acc_obj, acc_cls, acc_conf, acc_ncm, acc_nobj = carry0
    out_ref[0] = jnp.concatenate(
        [acc_obj, acc_cls, acc_conf, acc_ncm, acc_nobj], axis=0)


def kernel(x, target):
    nB = x.shape[0]
    nT = target.shape[1]
    xr = x.reshape(nB, _NA * (7 + _NC), _NCHUNK, _CHUNK)
    tp = jnp.pad(target, ((0, 0), (0, _TPAD - nT), (0, 1)))
    out = pl.pallas_call(
        _region_loss_kernel,
        grid=(nB,),
        in_specs=[
            pl.BlockSpec((1, _NA * (7 + _NC), _NCHUNK, _CHUNK),
                         lambda b: (b, 0, 0, 0)),
            pl.BlockSpec((1, _TPAD, 8), lambda b: (b, 0, 0)),
        ],
        out_specs=pl.BlockSpec((1, 5, _CHUNK), lambda b: (b, 0, 0)),
        out_shape=jax.ShapeDtypeStruct((nB, 5, _CHUNK), jnp.float32),
    )(xr, tp)
    sums = jnp.sum(out, axis=(0, 2))
    n_obj = jnp.maximum(sums[4], 1.0)
    n_cm = jnp.maximum(sums[3], 1.0)
    return (sums[0] + sums[1]) / n_obj + sums[2] / n_cm


# speedup vs baseline: 1.4043x; 1.0012x over previous
import jax
import jax.numpy as jnp
from jax.experimental import pallas as pl

_ANCHORS = ((1.08, 1.19), (3.42, 4.41), (6.63, 11.38), (9.42, 5.11), (16.62, 10.52))
_NA = 5
_NC = 7
_THR = 0.6
_H = 48
_W = 48
_TPAD = 56
_CHUNK = 128
_NCHUNK = (_H * _W) // _CHUNK
_EPS = 1e-12


_HITC = _THR / (1.0 + _THR)

_DN = (((0,), (0,)), ((), ()))


def _region_loss_kernel(x_ref, t_ref, out_ref):
    t = t_ref[0]
    lab = t[:, 0:1]
    gx = t[:, 1:2] * float(_W)
    gy = t[:, 2:3] * float(_H)
    gw = t[:, 3:4] * float(_W)
    gl = t[:, 4:5] * float(_H)
    gim = t[:, 5:6]
    gre = t[:, 6:7]
    valid = t[:, 1:2] > 0.0
    validf = jnp.where(valid, 1.0, 0.0)
    gif = jnp.clip(jnp.floor(gx), 0.0, float(_W - 1))
    gjf = jnp.clip(jnp.floor(gy), 0.0, float(_H - 1))
    txv = gx - gif
    tyv = gy - gjf
    area_g = gw * gl

    best_v = jnp.full_like(gx, -1.0)
    best_n = jnp.zeros_like(gx)
    best_w = jnp.full_like(gx, _ANCHORS[0][0])
    best_h = jnp.full_like(gx, _ANCHORS[0][1])
    anch_iou = []
    for a, (aw, ah) in enumerate(_ANCHORS):
        inter = jnp.minimum(gw, aw) * jnp.minimum(gl, ah)
        iou = inter / (area_g + aw * ah - inter + 1e-16)
        anch_iou.append(iou)
        upd = iou > best_v
        best_v = jnp.where(upd, iou, best_v)
        best_n = jnp.where(upd, float(a), best_n)
        best_w = jnp.where(upd, aw, best_w)
        best_h = jnp.where(upd, ah, best_h)
    twv = jnp.log(gw / best_w + 1e-16)
    tlv = jnp.log(gl / best_h + 1e-16)

    labcl = jnp.clip(lab, 0.0, float(_NC - 1))
    tio_i = jax.lax.broadcasted_iota(jnp.int32, (_TPAD, 1), 0)
    pow2 = jax.lax.bitcast_convert_type((tio_i + 127) << 23, jnp.float32)
    dblpow = pow2 * 2.0

    ones_col = jnp.ones_like(gx)
    vals_cols = jnp.concatenate(
        [ones_col, txv, tyv, twv, tlv, gim, gre], axis=1)

    hw = gw * 0.5
    hh = gl * 0.5
    b1x1 = gx - hw
    b1x2 = gx + hw
    b1y1 = gy - hh
    b1y2 = gy + hh

    lane = jax.lax.broadcasted_iota(jnp.int32, (1, _CHUNK), 1).astype(jnp.float32)

    acc0 = jnp.zeros((1, _CHUNK), dtype=jnp.float32)
    carry0 = (acc0, acc0, acc0, acc0, acc0)

    pafs, a_colss = [], []
    for a in range(_NA):
        paf = jnp.where(jnp.logical_and(valid, best_n == float(a)), 1.0, 0.0)
        zff = jnp.where(jnp.logical_and(anch_iou[a] > _THR, valid), 1.0, 0.0)
        labf = [jnp.where(labcl == float(c), 1.0, 0.0) * paf for c in range(_NC)]
        pafs.append(paf)
        a_colss.append(jnp.concatenate([pow2 * paf, zff] + labf, axis=1))

    ca_g = _HITC * area_g

    amaps = []
    for a, (aw, ah) in enumerate(_ANCHORS):
        base_c = a * (7 + _NC)
        px18 = jax.nn.sigmoid(x_ref[0, base_c + 0])
        py18 = jax.nn.sigmoid(x_ref[0, base_c + 1])
        bw18 = jnp.exp(x_ref[0, base_c + 2]) * aw
        bh18 = jnp.exp(x_ref[0, base_c + 3]) * ah
        conf18 = jax.nn.sigmoid(x_ref[0, base_c + 6])
        bcep18 = -jnp.log(conf18 + _EPS)
        bcen18 = -jnp.log(1.0 - conf18 + _EPS)
        lt = [x_ref[0, base_c + 7 + c] for c in range(_NC)]
        mx = lt[0]
        for c in range(1, _NC):
            mx = jnp.maximum(mx, lt[c])
        sm = jnp.exp(lt[0] - mx)
        for c in range(1, _NC):
            sm = sm + jnp.exp(lt[c] - mx)
        lse18 = jnp.log(sm) + mx
        amaps.append((px18, py18, bw18, bh18, bcep18, bcen18, lse18))

    def chunk_contrib(k, carry):
        acc_obj, acc_cls, acc_conf, acc_ncm, acc_nobj = carry
        idx = float(k * _CHUNK) + lane
        jcell = jnp.floor(idx * (1.0 / float(_W)))
        icell = idx - jcell * float(_W)
        cellm = jnp.where(
            jnp.logical_and(gif == icell, gjf == jcell), 1.0, 0.0)
        for a, (aw, ah) in enumerate(_ANCHORS):
            paf = pafs[a]
            a_cols = a_colss[a]
            base_c = a * (7 + _NC)
            px18, py18, bw18, bh18, bcep18, bcen18, lse18 = amaps[a]
            px = px18[k:k + 1, :]
            py = py18[k:k + 1, :]
            pw = x_ref[0, base_c + 2, pl.ds(k, 1), :]
            ph = x_ref[0, base_c + 3, pl.ds(k, 1), :]
            pim = x_ref[0, base_c + 4, pl.ds(k, 1), :]
            pre = x_ref[0, base_c + 5, pl.ds(k, 1), :]

            bw = bw18[k:k + 1, :]
            bh = bh18[k:k + 1, :]
            bx = px + icell
            by = py + jcell
            b2x1 = bx - bw * 0.5
            b2x2 = bx + bw * 0.5
            b2y1 = by - bh * 0.5
            b2y2 = by + bh * 0.5
            a2 = bw * bh

            iw = jnp.maximum(jnp.minimum(b1x2, b2x2) - jnp.maximum(b1x1, b2x1), 0.0)
            ih = jnp.maximum(jnp.minimum(b1y2, b2y2) - jnp.maximum(b1y1, b2y1), 0.0)
            inter = iw * ih
            hitf = jnp.where(inter > ca_g + _HITC * a2, 1.0, 0.0)
            hitcnt = jax.lax.dot_general(validf, hitf, _DN,
                                         preferred_element_type=jnp.float32)

            mm1 = jax.lax.dot_general(a_cols, cellm, _DN,
                                      preferred_element_type=jnp.float32)
            s_pow = mm1[0:1, :]
            s_z = mm1[1:2, :]
            w = cellm * paf * jnp.where(dblpow > s_pow, 1.0, 0.0)
            mm2 = jax.lax.dot_general(vals_cols, w, _DN,
                                      preferred_element_type=jnp.float32)
            anym = mm2[0:1, :]

            d = px - mm2[1:2, :]
            err = d * d
            d = py - mm2[2:3, :]
            err = err + d * d
            d = pw - mm2[3:4, :]
            err = err + d * d
            d = ph - mm2[4:5, :]
            err = err + d * d
            d = pim - mm2[5:6, :]
            err = err + d * d
            d = pre - mm2[6:7, :]
            err = err + d * d
            acc_obj = acc_obj + anym * err

            lse = lse18[k:k + 1, :]
            for c in range(_NC):
                anyc = jnp.minimum(mm1[2 + c:3 + c, :], 1.0)
                logit = x_ref[0, base_c + 7 + c, pl.ds(k, 1), :]
                acc_cls = acc_cls + anyc * (lse - logit)

            base = jnp.where(hitcnt > 0.0, 0.0, 1.0)
            cm = jnp.where(anym > 0.0, 1.0, jnp.where(s_z > 0.0, 0.0, base))
            bce = jnp.where(anym > 0.0, bcep18[k:k + 1, :], bcen18[k:k + 1, :])
            acc_conf = acc_conf + cm * bce
            acc_ncm = acc_ncm + cm
            acc_nobj = acc_nobj + anym
        return (acc_obj, acc_cls, acc_conf, acc_ncm, acc_nobj)

    for kk in range(_NCHUNK):
        carry0 = chunk_contrib(kk, carry0)

    acc_obj, acc_cls, acc_conf, acc_ncm, acc_nobj = carry0
    out_ref[0] = jnp.concatenate(
        [acc_obj, acc_cls, acc_conf, acc_ncm, acc_nobj], axis=0)


def kernel(x, target):
    nB = x.shape[0]
    nT = target.shape[1]
    xr = x.reshape(nB, _NA * (7 + _NC), _NCHUNK, _CHUNK)
    tp = jnp.pad(target, ((0, 0), (0, _TPAD - nT), (0, 1)))
    out = pl.pallas_call(
        _region_loss_kernel,
        grid=(nB,),
        in_specs=[
            pl.BlockSpec((1, _NA * (7 + _NC), _NCHUNK, _CHUNK),
                         lambda b: (b, 0, 0, 0)),
            pl.BlockSpec((1, _TPAD, 8), lambda b: (b, 0, 0)),
        ],
        out_specs=pl.BlockSpec((1, 5, _CHUNK), lambda b: (b, 0, 0)),
        out_shape=jax.ShapeDtypeStruct((nB, 5, _CHUNK), jnp.float32),
    )(xr, tp)
    sums = jnp.sum(out, axis=(0, 2))
    n_obj = jnp.maximum(sums[4], 1.0)
    n_cm = jnp.maximum(sums[3], 1.0)
    return (sums[0] + sums[1]) / n_obj + sums[2] / n_cm
